# Initial kernel scaffold; baseline (speedup 1.0000x reference)
import jax
import jax.numpy as jnp
from jax.experimental import pallas as pl


def kernel(x, pos_table):
    raise NotImplementedError("write your pallas kernel here")


# speedup vs baseline: 4.4582x; 4.4582x over previous
import jax
import jax.numpy as jnp
from jax.experimental import pallas as pl

BT = 256


def _add_pos_kernel(x_ref, pos_ref, out_ref):
    out_ref[...] = x_ref[...] + pos_ref[...][:, None, :]


def kernel(x, pos_table):
    T, B, D = x.shape
    grid = (T // BT,)
    return pl.pallas_call(
        _add_pos_kernel,
        grid=grid,
        in_specs=[
            pl.BlockSpec((BT, B, D), lambda i: (i, 0, 0)),
            pl.BlockSpec((BT, D), lambda i: (i, 0)),
        ],
        out_specs=pl.BlockSpec((BT, B, D), lambda i: (i, 0, 0)),
        out_shape=jax.ShapeDtypeStruct((T, B, D), x.dtype),
    )(x, pos_table)


# speedup vs baseline: 4.5288x; 1.0158x over previous
import jax
import jax.numpy as jnp
from jax.experimental import pallas as pl

BT = 512


def _add_pos_kernel(x_ref, pos_ref, out_ref):
    out_ref[...] = x_ref[...] + pos_ref[...][:, None, :]


def kernel(x, pos_table):
    T, B, D = x.shape
    grid = (T // BT,)
    return pl.pallas_call(
        _add_pos_kernel,
        grid=grid,
        in_specs=[
            pl.BlockSpec((BT, B, D), lambda i: (i, 0, 0)),
            pl.BlockSpec((BT, D), lambda i: (i, 0)),
        ],
        out_specs=pl.BlockSpec((BT, B, D), lambda i: (i, 0, 0)),
        out_shape=jax.ShapeDtypeStruct((T, B, D), x.dtype),
    )(x, pos_table)
